# Initial kernel scaffold; baseline (speedup 1.0000x reference)
import jax
import jax.numpy as jnp
from jax.experimental import pallas as pl


def kernel(company_emb, lap0_idx, lap0_val, lap1_idx, lap1_val, lap2_idx, lap2_val, W, alpha):
    raise NotImplementedError("write your pallas kernel here")


# speedup vs baseline: 4.2418x; 4.2418x over previous
import functools

import numpy as np
import jax
import jax.numpy as jnp
from jax import lax
from jax.experimental import pallas as pl
from jax.experimental.pallas import tpu as pltpu
from jax.experimental.pallas import tpu_sc as plsc

N = 10000
D = 128
NNZ = 320000
NUM_HG = 3
E = NUM_HG * NNZ
CHUNK = 128
NUM_CHUNKS = E // CHUNK
NC = 2
NS = 16
L = 16
NPAD = 10240
ROWS_PER_TILE = NPAD // NS
ZROWS = 128


def _proj_body(x_ref, w_ref, vals_ref, alpha_ref, h_ref, sv_ref):
    h_ref[...] = lax.dot_general(
        x_ref[...], w_ref[...], (((1,), (1,)), ((), ())),
        preferred_element_type=jnp.float32)
    a = jax.nn.sigmoid(alpha_ref[...])
    sv_ref[...] = vals_ref[...] * a


def _project_and_scale(x, w, vals3, alpha):
    return pl.pallas_call(
        _proj_body,
        grid=(10,),
        in_specs=[
            pl.BlockSpec((N // 10, D), lambda i: (i, 0)),
            pl.BlockSpec((D, D), lambda i: (0, 0)),
            pl.BlockSpec((NUM_HG, NNZ // 10), lambda i: (0, i)),
            pl.BlockSpec((NUM_HG, 1), lambda i: (0, 0)),
        ],
        out_specs=[
            pl.BlockSpec((N // 10, D), lambda i: (i, 0)),
            pl.BlockSpec((NUM_HG, NNZ // 10), lambda i: (0, i)),
        ],
        out_shape=[
            jax.ShapeDtypeStruct((N, D), jnp.float32),
            jax.ShapeDtypeStruct((NUM_HG, NNZ), jnp.float32),
        ],
    )(x, w, vals3, alpha)


_mesh = plsc.VectorSubcoreMesh(core_axis_name="c", subcore_axis_name="s")


@functools.partial(
    pl.kernel,
    out_type=jax.ShapeDtypeStruct((NC, NPAD, D), jnp.float32),
    mesh=_mesh,
    scratch_types=[
        pltpu.VMEM_SHARED((NPAD, D), jnp.float32),
        pltpu.VMEM((CHUNK,), jnp.int32),
        pltpu.VMEM((CHUNK,), jnp.int32),
        pltpu.VMEM((CHUNK,), jnp.float32),
        pltpu.VMEM((CHUNK, D), jnp.float32),
        pltpu.VMEM((ZROWS, D), jnp.float32),
        pltpu.SemaphoreType.DMA,
    ],
)
def _spmm_kernel(h_hbm, cols_hbm, rows_hbm, vals_hbm, out_hbm,
                 acc, cidx, ridx, valv, rows_buf, zbuf, gsem):
    cid = lax.axis_index("c")
    sid = lax.axis_index("s")
    w = cid * NS + sid

    zeros16 = jnp.zeros((L,), jnp.float32)

    @pl.loop(0, ZROWS)
    def _zero_zbuf(i):
        for j in range(D // L):
            zbuf[i, pl.ds(j * L, L)] = zeros16

    @pl.loop(0, ROWS_PER_TILE // ZROWS)
    def _zero_acc(t):
        pltpu.sync_copy(zbuf, acc.at[pl.ds(sid * ROWS_PER_TILE + t * ZROWS, ZROWS)])

    plsc.subcore_barrier()

    @pl.loop(w, NUM_CHUNKS, step=NC * NS)
    def _chunk(c):
        base = c * CHUNK
        pltpu.sync_copy(cols_hbm.at[pl.ds(base, CHUNK)], cidx)
        pltpu.sync_copy(rows_hbm.at[pl.ds(base, CHUNK)], ridx)
        pltpu.sync_copy(vals_hbm.at[pl.ds(base, CHUNK)], valv)
        pltpu.async_copy(h_hbm.at[cidx], rows_buf, gsem).wait()

        @plsc.parallel_loop(0, CHUNK // L)
        def _grp(g):
            vv = valv[pl.ds(g * L, L)]
            for l in range(L):
                vs = jnp.broadcast_to(vv[l], (L,))
                e = g * L + l
                for j in range(D // L):
                    sl = pl.ds(j * L, L)
                    rows_buf[e, sl] = rows_buf[e, sl] * vs

        pltpu.sync_copy(rows_buf, acc.at[ridx], add=True)

    plsc.subcore_barrier()

    r0 = sid * ROWS_PER_TILE
    pltpu.sync_copy(acc.at[pl.ds(r0, ROWS_PER_TILE)],
                    out_hbm.at[cid].at[pl.ds(r0, ROWS_PER_TILE)])


def _combine_body(p_ref, o_ref):
    o_ref[...] = p_ref[0] + p_ref[1]


def _combine(partials):
    return pl.pallas_call(
        _combine_body,
        grid=(10,),
        in_specs=[pl.BlockSpec((NC, N // 10, D), lambda i: (0, i, 0))],
        out_specs=pl.BlockSpec((N // 10, D), lambda i: (i, 0)),
        out_shape=jax.ShapeDtypeStruct((N, D), jnp.float32),
    )(partials)


def kernel(company_emb, lap0_idx, lap0_val, lap1_idx, lap1_val, lap2_idx,
           lap2_val, W, alpha):
    vals3 = jnp.stack([lap0_val, lap1_val, lap2_val])
    h, vals_scaled = _project_and_scale(company_emb, W, vals3, alpha)
    cols = jnp.concatenate(
        [lap0_idx[1], lap1_idx[1], lap2_idx[1]]).astype(jnp.int32)
    rows = jnp.concatenate(
        [lap0_idx[0], lap1_idx[0], lap2_idx[0]]).astype(jnp.int32)
    partials = _spmm_kernel(h, cols, rows, vals_scaled.reshape(-1))
    return _combine(partials)
